# Initial kernel scaffold; baseline (speedup 1.0000x reference)
import jax
import jax.numpy as jnp
from jax.experimental import pallas as pl


def kernel(Z, H, noise, sigmas, W_enc, W_pos, W_out, W1, b1, W2, b2, block_id, batch_id, noise_level):
    raise NotImplementedError("write your pallas kernel here")


# speedup vs baseline: 1.8600x; 1.8600x over previous
import jax
import jax.numpy as jnp
from jax.experimental import pallas as pl
from jax.experimental.pallas import tpu as pltpu

N_A = 320000
N_B = 32000
N_G = 64
HID = 128
BB = 256
AC = 1024
N_T = N_B // BB


def _silu(x):
    return x * jax.nn.sigmoid(x)


def _fused_kernel(starts_ref,
                  h_hbm, z_hbm, bid_hbm,
                  nsig_ref, noise_ref, batch_ref,
                  wenc_ref, wpos_ref, wout_ref, w1_ref, b1_ref, w2_ref, b2_ref,
                  energy_ref, graph_ref, loss_ref,
                  sums_ref, cnts_ref, hbuf, zbuf, bbuf,
                  gsum_ref, gcnt_ref, losss_ref,
                  sem_h, sem_z, sem_b):
    t = pl.program_id(0)
    b0 = t * BB
    astart = starts_ref[t]
    aend = starts_ref[t + 1]

    @pl.when(t == 0)
    def _():
        gsum_ref[...] = jnp.zeros_like(gsum_ref)
        gcnt_ref[...] = jnp.zeros_like(gcnt_ref)
        losss_ref[...] = jnp.zeros_like(losss_ref)

    sums_ref[...] = jnp.zeros_like(sums_ref)
    cnts_ref[...] = jnp.zeros_like(cnts_ref)

    n_sub = (aend - astart + AC - 1) // AC

    iota_bb = jax.lax.broadcasted_iota(jnp.int32, (AC, BB), 1)
    row_iota = jax.lax.broadcasted_iota(jnp.int32, (AC, 1), 0)
    ones_a = jnp.ones((AC, 1), jnp.float32)

    def body(i, carry):
        base = astart + i * AC
        base_c = jnp.minimum(base, N_A - AC)
        ch = pltpu.make_async_copy(h_hbm.at[pl.ds(base_c, AC), :], hbuf, sem_h)
        cz = pltpu.make_async_copy(z_hbm.at[pl.ds(base_c, AC), :], zbuf, sem_z)
        cb = pltpu.make_async_copy(bid_hbm.at[pl.ds(base_c, AC), :], bbuf, sem_b)
        ch.start()
        cz.start()
        cb.start()
        ch.wait()
        cz.wait()
        cb.wait()
        local = bbuf[...] - b0
        valid = (base_c + row_iota) >= base
        oh = jnp.where(jnp.logical_and(local == iota_bb, valid),
                       1.0, 0.0).astype(jnp.float32)
        nadd = jax.lax.dot_general(oh, nsig_ref[...], (((1,), (0,)), ((), ())),
                                   preferred_element_type=jnp.float32, precision=jax.lax.Precision.HIGHEST)
        zp = zbuf[...] + nadd
        pre = jnp.dot(hbuf[...], wenc_ref[...], preferred_element_type=jnp.float32, precision=jax.lax.Precision.HIGHEST)
        pre = (pre + zp[:, 0:1] * wpos_ref[0:1, :]
               + zp[:, 1:2] * wpos_ref[1:2, :]
               + zp[:, 2:3] * wpos_ref[2:3, :])
        u = _silu(pre)
        sums_ref[...] += jax.lax.dot_general(oh, u, (((0,), (0,)), ((), ())),
                                             preferred_element_type=jnp.float32, precision=jax.lax.Precision.HIGHEST)
        cnts_ref[...] += jax.lax.dot_general(oh, ones_a, (((0,), (0,)), ((), ())),
                                             preferred_element_type=jnp.float32, precision=jax.lax.Precision.HIGHEST)
        return carry

    jax.lax.fori_loop(0, n_sub, body, 0)

    cnt = cnts_ref[...]
    block_repr = sums_ref[...] / jnp.maximum(cnt, 1.0)
    iota_g = jax.lax.broadcasted_iota(jnp.int32, (BB, N_G), 1)
    ohg = jnp.where(batch_ref[...] == iota_g, 1.0, 0.0).astype(jnp.float32)
    gsum_ref[...] += jax.lax.dot_general(ohg, block_repr, (((0,), (0,)), ((), ())),
                                         preferred_element_type=jnp.float32, precision=jax.lax.Precision.HIGHEST)
    gcnt_ref[...] += jax.lax.dot_general(ohg, jnp.ones((BB, 1), jnp.float32),
                                         (((0,), (0,)), ((), ())),
                                         preferred_element_type=jnp.float32, precision=jax.lax.Precision.HIGHEST)
    pred = jnp.dot(block_repr, wout_ref[...], preferred_element_type=jnp.float32, precision=jax.lax.Precision.HIGHEST)
    score = -noise_ref[...] * jnp.where(cnt > 0.0, 1.0, 0.0)
    losss_ref[...] += jnp.reshape(jnp.sum((pred - score) ** 2), (1, 1))

    graph_repr = gsum_ref[...] / jnp.maximum(gcnt_ref[...], 1.0)
    graph_ref[...] = graph_repr
    e1 = _silu(jnp.dot(graph_repr, w1_ref[...], preferred_element_type=jnp.float32, precision=jax.lax.Precision.HIGHEST)
               + b1_ref[...])
    energy_ref[...] = jnp.dot(e1, w2_ref[...],
                              preferred_element_type=jnp.float32, precision=jax.lax.Precision.HIGHEST) + b2_ref[...]
    loss_ref[...] = losss_ref[...] / (N_B * 3.0)


def kernel(Z, H, noise, sigmas, W_enc, W_pos, W_out, W1, b1, W2, b2,
           block_id, batch_id, noise_level):
    sig = sigmas[noise_level][batch_id]
    nsig = noise * sig[:, None]
    bounds = jnp.arange(N_T + 1, dtype=jnp.int32) * BB
    starts = jnp.searchsorted(block_id, bounds, side='left').astype(jnp.int32)
    bid2d = block_id.reshape(N_A, 1)
    batch2d = batch_id.reshape(N_B, 1)

    grid_spec = pltpu.PrefetchScalarGridSpec(
        num_scalar_prefetch=1,
        grid=(N_T,),
        in_specs=[
            pl.BlockSpec(memory_space=pl.ANY),
            pl.BlockSpec(memory_space=pl.ANY),
            pl.BlockSpec(memory_space=pl.ANY),
            pl.BlockSpec((BB, 3), lambda t, s: (t, 0)),
            pl.BlockSpec((BB, 3), lambda t, s: (t, 0)),
            pl.BlockSpec((BB, 1), lambda t, s: (t, 0)),
            pl.BlockSpec((HID, HID), lambda t, s: (0, 0)),
            pl.BlockSpec((3, HID), lambda t, s: (0, 0)),
            pl.BlockSpec((HID, 3), lambda t, s: (0, 0)),
            pl.BlockSpec((HID, HID), lambda t, s: (0, 0)),
            pl.BlockSpec((1, HID), lambda t, s: (0, 0)),
            pl.BlockSpec((HID, 1), lambda t, s: (0, 0)),
            pl.BlockSpec((1, 1), lambda t, s: (0, 0)),
        ],
        out_specs=[
            pl.BlockSpec((N_G, 1), lambda t, s: (0, 0)),
            pl.BlockSpec((N_G, HID), lambda t, s: (0, 0)),
            pl.BlockSpec((1, 1), lambda t, s: (0, 0)),
        ],
        scratch_shapes=[
            pltpu.VMEM((BB, HID), jnp.float32),
            pltpu.VMEM((BB, 1), jnp.float32),
            pltpu.VMEM((AC, HID), jnp.float32),
            pltpu.VMEM((AC, 3), jnp.float32),
            pltpu.VMEM((AC, 1), jnp.int32),
            pltpu.VMEM((N_G, HID), jnp.float32),
            pltpu.VMEM((N_G, 1), jnp.float32),
            pltpu.VMEM((1, 1), jnp.float32),
            pltpu.SemaphoreType.DMA,
            pltpu.SemaphoreType.DMA,
            pltpu.SemaphoreType.DMA,
        ],
    )
    energy, graph_repr, loss = pl.pallas_call(
        _fused_kernel,
        grid_spec=grid_spec,
        out_shape=[
            jax.ShapeDtypeStruct((N_G, 1), jnp.float32),
            jax.ShapeDtypeStruct((N_G, HID), jnp.float32),
            jax.ShapeDtypeStruct((1, 1), jnp.float32),
        ],
        compiler_params=pltpu.CompilerParams(
            dimension_semantics=("arbitrary",)),
    )(starts, H, Z, bid2d, nsig, noise, batch2d,
      W_enc, W_pos, W_out, W1, b1.reshape(1, HID), W2, b2.reshape(1, 1))
    return energy[:, 0], graph_repr, loss[0, 0]


# speedup vs baseline: 2.1377x; 1.1493x over previous
import jax
import jax.numpy as jnp
from jax.experimental import pallas as pl
from jax.experimental.pallas import tpu as pltpu

N_A = 320000
N_B = 32000
N_G = 64
HID = 128
BB = 256
AC = 1024
N_T = N_B // BB


def _silu(x):
    return x * jax.nn.sigmoid(x)


def _fused_kernel(starts_ref,
                  h_hbm, z_hbm, bid_hbm,
                  nsig_ref, noise_ref, batch_ref,
                  wenc_ref, wpos_ref, wout_ref, w1_ref, b1_ref, w2_ref, b2_ref,
                  energy_ref, graph_ref, loss_ref,
                  sums_ref, cnts_ref, hbuf, zbuf, bbuf,
                  gsum_ref, gcnt_ref, losss_ref,
                  sem_h, sem_z, sem_b):
    t = pl.program_id(0)
    b0 = t * BB
    astart = starts_ref[t]
    aend = starts_ref[t + 1]

    @pl.when(t == 0)
    def _():
        gsum_ref[...] = jnp.zeros_like(gsum_ref)
        gcnt_ref[...] = jnp.zeros_like(gcnt_ref)
        losss_ref[...] = jnp.zeros_like(losss_ref)

    sums_ref[...] = jnp.zeros_like(sums_ref)
    cnts_ref[...] = jnp.zeros_like(cnts_ref)

    n_sub = (aend - astart + AC - 1) // AC

    iota_bb = jax.lax.broadcasted_iota(jnp.int32, (AC, BB), 1)
    row_iota = jax.lax.broadcasted_iota(jnp.int32, (AC, 1), 0)
    ones_a = jnp.ones((AC, 1), jnp.float32)

    def start_copies(i, slot):
        base = astart + i * AC
        base_c = jnp.minimum(base, N_A - AC)
        pltpu.make_async_copy(h_hbm.at[pl.ds(base_c, AC), :],
                              hbuf.at[slot], sem_h.at[slot]).start()
        pltpu.make_async_copy(z_hbm.at[pl.ds(base_c, AC), :],
                              zbuf.at[slot], sem_z.at[slot]).start()
        pltpu.make_async_copy(bid_hbm.at[pl.ds(base_c, AC), :],
                              bbuf.at[slot], sem_b.at[slot]).start()

    def wait_copies(i, slot):
        base = astart + i * AC
        base_c = jnp.minimum(base, N_A - AC)
        pltpu.make_async_copy(h_hbm.at[pl.ds(base_c, AC), :],
                              hbuf.at[slot], sem_h.at[slot]).wait()
        pltpu.make_async_copy(z_hbm.at[pl.ds(base_c, AC), :],
                              zbuf.at[slot], sem_z.at[slot]).wait()
        pltpu.make_async_copy(bid_hbm.at[pl.ds(base_c, AC), :],
                              bbuf.at[slot], sem_b.at[slot]).wait()

    @pl.when(n_sub > 0)
    def _():
        start_copies(0, 0)

    def body(i, carry):
        slot = jax.lax.rem(i, 2)
        base = astart + i * AC
        base_c = jnp.minimum(base, N_A - AC)

        @pl.when(i + 1 < n_sub)
        def _():
            start_copies(i + 1, 1 - slot)

        wait_copies(i, slot)
        local = bbuf[slot] - b0
        valid = (base_c + row_iota) >= base
        oh = jnp.where(jnp.logical_and(local == iota_bb, valid),
                       1.0, 0.0).astype(jnp.float32)
        nadd = jax.lax.dot_general(oh, nsig_ref[...], (((1,), (0,)), ((), ())),
                                   preferred_element_type=jnp.float32, precision=jax.lax.Precision.HIGHEST)
        zp = zbuf[slot] + nadd
        pre = jnp.dot(hbuf[slot], wenc_ref[...], preferred_element_type=jnp.float32, precision=jax.lax.Precision.HIGHEST)
        pre = (pre + zp[:, 0:1] * wpos_ref[0:1, :]
               + zp[:, 1:2] * wpos_ref[1:2, :]
               + zp[:, 2:3] * wpos_ref[2:3, :])
        u = _silu(pre)
        sums_ref[...] += jax.lax.dot_general(oh, u, (((0,), (0,)), ((), ())),
                                             preferred_element_type=jnp.float32, precision=jax.lax.Precision.HIGHEST)
        cnts_ref[...] += jax.lax.dot_general(oh, ones_a, (((0,), (0,)), ((), ())),
                                             preferred_element_type=jnp.float32, precision=jax.lax.Precision.HIGHEST)
        return carry

    jax.lax.fori_loop(0, n_sub, body, 0)

    cnt = cnts_ref[...]
    block_repr = sums_ref[...] / jnp.maximum(cnt, 1.0)
    iota_g = jax.lax.broadcasted_iota(jnp.int32, (BB, N_G), 1)
    ohg = jnp.where(batch_ref[...] == iota_g, 1.0, 0.0).astype(jnp.float32)
    gsum_ref[...] += jax.lax.dot_general(ohg, block_repr, (((0,), (0,)), ((), ())),
                                         preferred_element_type=jnp.float32, precision=jax.lax.Precision.HIGHEST)
    gcnt_ref[...] += jax.lax.dot_general(ohg, jnp.ones((BB, 1), jnp.float32),
                                         (((0,), (0,)), ((), ())),
                                         preferred_element_type=jnp.float32, precision=jax.lax.Precision.HIGHEST)
    pred = jnp.dot(block_repr, wout_ref[...], preferred_element_type=jnp.float32, precision=jax.lax.Precision.HIGHEST)
    score = -noise_ref[...] * jnp.where(cnt > 0.0, 1.0, 0.0)
    losss_ref[...] += jnp.reshape(jnp.sum((pred - score) ** 2), (1, 1))

    graph_repr = gsum_ref[...] / jnp.maximum(gcnt_ref[...], 1.0)
    graph_ref[...] = graph_repr
    e1 = _silu(jnp.dot(graph_repr, w1_ref[...], preferred_element_type=jnp.float32, precision=jax.lax.Precision.HIGHEST)
               + b1_ref[...])
    energy_ref[...] = jnp.dot(e1, w2_ref[...],
                              preferred_element_type=jnp.float32, precision=jax.lax.Precision.HIGHEST) + b2_ref[...]
    loss_ref[...] = losss_ref[...] / (N_B * 3.0)


def kernel(Z, H, noise, sigmas, W_enc, W_pos, W_out, W1, b1, W2, b2,
           block_id, batch_id, noise_level):
    sig = sigmas[noise_level][batch_id]
    nsig = noise * sig[:, None]
    bounds = jnp.arange(N_T + 1, dtype=jnp.int32) * BB
    starts = jnp.searchsorted(block_id, bounds, side='left').astype(jnp.int32)
    bid2d = block_id.reshape(N_A, 1)
    batch2d = batch_id.reshape(N_B, 1)

    grid_spec = pltpu.PrefetchScalarGridSpec(
        num_scalar_prefetch=1,
        grid=(N_T,),
        in_specs=[
            pl.BlockSpec(memory_space=pl.ANY),
            pl.BlockSpec(memory_space=pl.ANY),
            pl.BlockSpec(memory_space=pl.ANY),
            pl.BlockSpec((BB, 3), lambda t, s: (t, 0)),
            pl.BlockSpec((BB, 3), lambda t, s: (t, 0)),
            pl.BlockSpec((BB, 1), lambda t, s: (t, 0)),
            pl.BlockSpec((HID, HID), lambda t, s: (0, 0)),
            pl.BlockSpec((3, HID), lambda t, s: (0, 0)),
            pl.BlockSpec((HID, 3), lambda t, s: (0, 0)),
            pl.BlockSpec((HID, HID), lambda t, s: (0, 0)),
            pl.BlockSpec((1, HID), lambda t, s: (0, 0)),
            pl.BlockSpec((HID, 1), lambda t, s: (0, 0)),
            pl.BlockSpec((1, 1), lambda t, s: (0, 0)),
        ],
        out_specs=[
            pl.BlockSpec((N_G, 1), lambda t, s: (0, 0)),
            pl.BlockSpec((N_G, HID), lambda t, s: (0, 0)),
            pl.BlockSpec((1, 1), lambda t, s: (0, 0)),
        ],
        scratch_shapes=[
            pltpu.VMEM((BB, HID), jnp.float32),
            pltpu.VMEM((BB, 1), jnp.float32),
            pltpu.VMEM((2, AC, HID), jnp.float32),
            pltpu.VMEM((2, AC, 3), jnp.float32),
            pltpu.VMEM((2, AC, 1), jnp.int32),
            pltpu.VMEM((N_G, HID), jnp.float32),
            pltpu.VMEM((N_G, 1), jnp.float32),
            pltpu.VMEM((1, 1), jnp.float32),
            pltpu.SemaphoreType.DMA((2,)),
            pltpu.SemaphoreType.DMA((2,)),
            pltpu.SemaphoreType.DMA((2,)),
        ],
    )
    energy, graph_repr, loss = pl.pallas_call(
        _fused_kernel,
        grid_spec=grid_spec,
        out_shape=[
            jax.ShapeDtypeStruct((N_G, 1), jnp.float32),
            jax.ShapeDtypeStruct((N_G, HID), jnp.float32),
            jax.ShapeDtypeStruct((1, 1), jnp.float32),
        ],
        compiler_params=pltpu.CompilerParams(
            dimension_semantics=("arbitrary",)),
    )(starts, H, Z, bid2d, nsig, noise, batch2d,
      W_enc, W_pos, W_out, W1, b1.reshape(1, HID), W2, b2.reshape(1, 1))
    return energy[:, 0], graph_repr, loss[0, 0]


# speedup vs baseline: 2.6277x; 1.2292x over previous
import jax
import jax.numpy as jnp
from jax.experimental import pallas as pl
from jax.experimental.pallas import tpu as pltpu

N_A = 320000
N_B = 32000
N_G = 64
HID = 128
BB = 256
AC = 1024
N_T = N_B // BB


def _silu(x):
    return x * jax.nn.sigmoid(x)


def _fused_kernel(starts_ref,
                  h_hbm, z_hbm, bid_hbm,
                  nsig_ref, noise_ref, batch_ref,
                  wenc_ref, wpos_ref, wout_ref, w1_ref, b1_ref, w2_ref, b2_ref,
                  energy_ref, graph_ref, loss_ref,
                  sums_ref, cnts_ref, hbuf, zbuf, bbuf,
                  gsum_ref, gcnt_ref, losss_ref,
                  sem_h, sem_z, sem_b):
    t = pl.program_id(0)
    b0 = t * BB
    astart = starts_ref[t]
    aend = starts_ref[t + 1]

    @pl.when(t == 0)
    def _():
        gsum_ref[...] = jnp.zeros_like(gsum_ref)
        gcnt_ref[...] = jnp.zeros_like(gcnt_ref)
        losss_ref[...] = jnp.zeros_like(losss_ref)

    sums_ref[...] = jnp.zeros_like(sums_ref)
    cnts_ref[...] = jnp.zeros_like(cnts_ref)

    n_sub = (aend - astart + AC - 1) // AC

    iota_bb = jax.lax.broadcasted_iota(jnp.int32, (AC, BB), 1)
    row_iota = jax.lax.broadcasted_iota(jnp.int32, (AC, 1), 0)
    ones_a = jnp.ones((AC, 1), jnp.float32)

    def start_copies(i, slot):
        base = astart + i * AC
        base_c = jnp.minimum(base, N_A - AC)
        pltpu.make_async_copy(h_hbm.at[pl.ds(base_c, AC), :],
                              hbuf.at[slot], sem_h.at[slot]).start()
        pltpu.make_async_copy(z_hbm.at[pl.ds(base_c, AC), :],
                              zbuf.at[slot], sem_z.at[slot]).start()
        pltpu.make_async_copy(bid_hbm.at[pl.ds(base_c, AC), :],
                              bbuf.at[slot], sem_b.at[slot]).start()

    def wait_copies(i, slot):
        base = astart + i * AC
        base_c = jnp.minimum(base, N_A - AC)
        pltpu.make_async_copy(h_hbm.at[pl.ds(base_c, AC), :],
                              hbuf.at[slot], sem_h.at[slot]).wait()
        pltpu.make_async_copy(z_hbm.at[pl.ds(base_c, AC), :],
                              zbuf.at[slot], sem_z.at[slot]).wait()
        pltpu.make_async_copy(bid_hbm.at[pl.ds(base_c, AC), :],
                              bbuf.at[slot], sem_b.at[slot]).wait()

    @pl.when(n_sub > 0)
    def _():
        start_copies(0, 0)

    def body(i, carry):
        slot = jax.lax.rem(i, 2)
        base = astart + i * AC
        base_c = jnp.minimum(base, N_A - AC)

        @pl.when(i + 1 < n_sub)
        def _():
            start_copies(i + 1, 1 - slot)

        wait_copies(i, slot)
        local = bbuf[slot] - b0
        valid = (base_c + row_iota) >= base
        oh = jnp.where(jnp.logical_and(local == iota_bb, valid),
                       1.0, 0.0).astype(jnp.float32)
        nadd = jax.lax.dot_general(oh, nsig_ref[...], (((1,), (0,)), ((), ())),
                                   preferred_element_type=jnp.float32, precision=jax.lax.Precision.HIGHEST)
        zp = zbuf[slot] + nadd
        pre = jnp.dot(hbuf[slot].astype(jnp.bfloat16),
                      wenc_ref[...].astype(jnp.bfloat16),
                      preferred_element_type=jnp.float32)
        zp16 = zp.astype(jnp.bfloat16).astype(jnp.float32)
        wpos16 = wpos_ref[...].astype(jnp.bfloat16).astype(jnp.float32)
        pre = (pre + zp16[:, 0:1] * wpos16[0:1, :]
               + zp16[:, 1:2] * wpos16[1:2, :]
               + zp16[:, 2:3] * wpos16[2:3, :])
        u = _silu(pre)
        sums_ref[...] += jax.lax.dot_general(oh, u, (((0,), (0,)), ((), ())),
                                             preferred_element_type=jnp.float32, precision=jax.lax.Precision.HIGHEST)
        cnts_ref[...] += jax.lax.dot_general(oh, ones_a, (((0,), (0,)), ((), ())),
                                             preferred_element_type=jnp.float32, precision=jax.lax.Precision.HIGHEST)
        return carry

    jax.lax.fori_loop(0, n_sub, body, 0)

    cnt = cnts_ref[...]
    block_repr = sums_ref[...] / jnp.maximum(cnt, 1.0)
    iota_g = jax.lax.broadcasted_iota(jnp.int32, (BB, N_G), 1)
    ohg = jnp.where(batch_ref[...] == iota_g, 1.0, 0.0).astype(jnp.float32)
    gsum_ref[...] += jax.lax.dot_general(ohg, block_repr, (((0,), (0,)), ((), ())),
                                         preferred_element_type=jnp.float32, precision=jax.lax.Precision.HIGHEST)
    gcnt_ref[...] += jax.lax.dot_general(ohg, jnp.ones((BB, 1), jnp.float32),
                                         (((0,), (0,)), ((), ())),
                                         preferred_element_type=jnp.float32, precision=jax.lax.Precision.HIGHEST)
    pred = jnp.dot(block_repr, wout_ref[...], preferred_element_type=jnp.float32, precision=jax.lax.Precision.HIGHEST)
    score = -noise_ref[...] * jnp.where(cnt > 0.0, 1.0, 0.0)
    losss_ref[...] += jnp.reshape(jnp.sum((pred - score) ** 2), (1, 1))

    graph_repr = gsum_ref[...] / jnp.maximum(gcnt_ref[...], 1.0)
    graph_ref[...] = graph_repr
    e1 = _silu(jnp.dot(graph_repr.astype(jnp.bfloat16),
                       w1_ref[...].astype(jnp.bfloat16),
                       preferred_element_type=jnp.float32) + b1_ref[...])
    energy_ref[...] = jnp.dot(e1.astype(jnp.bfloat16),
                              w2_ref[...].astype(jnp.bfloat16),
                              preferred_element_type=jnp.float32) + b2_ref[...]
    loss_ref[...] = losss_ref[...] / (N_B * 3.0)


def kernel(Z, H, noise, sigmas, W_enc, W_pos, W_out, W1, b1, W2, b2,
           block_id, batch_id, noise_level):
    sig = sigmas[noise_level][batch_id]
    nsig = noise * sig[:, None]
    bounds = jnp.arange(N_T + 1, dtype=jnp.int32) * BB
    starts = jnp.searchsorted(block_id, bounds, side='left').astype(jnp.int32)
    bid2d = block_id.reshape(N_A, 1)
    batch2d = batch_id.reshape(N_B, 1)

    grid_spec = pltpu.PrefetchScalarGridSpec(
        num_scalar_prefetch=1,
        grid=(N_T,),
        in_specs=[
            pl.BlockSpec(memory_space=pl.ANY),
            pl.BlockSpec(memory_space=pl.ANY),
            pl.BlockSpec(memory_space=pl.ANY),
            pl.BlockSpec((BB, 3), lambda t, s: (t, 0)),
            pl.BlockSpec((BB, 3), lambda t, s: (t, 0)),
            pl.BlockSpec((BB, 1), lambda t, s: (t, 0)),
            pl.BlockSpec((HID, HID), lambda t, s: (0, 0)),
            pl.BlockSpec((3, HID), lambda t, s: (0, 0)),
            pl.BlockSpec((HID, 3), lambda t, s: (0, 0)),
            pl.BlockSpec((HID, HID), lambda t, s: (0, 0)),
            pl.BlockSpec((1, HID), lambda t, s: (0, 0)),
            pl.BlockSpec((HID, 1), lambda t, s: (0, 0)),
            pl.BlockSpec((1, 1), lambda t, s: (0, 0)),
        ],
        out_specs=[
            pl.BlockSpec((N_G, 1), lambda t, s: (0, 0)),
            pl.BlockSpec((N_G, HID), lambda t, s: (0, 0)),
            pl.BlockSpec((1, 1), lambda t, s: (0, 0)),
        ],
        scratch_shapes=[
            pltpu.VMEM((BB, HID), jnp.float32),
            pltpu.VMEM((BB, 1), jnp.float32),
            pltpu.VMEM((2, AC, HID), jnp.float32),
            pltpu.VMEM((2, AC, 3), jnp.float32),
            pltpu.VMEM((2, AC, 1), jnp.int32),
            pltpu.VMEM((N_G, HID), jnp.float32),
            pltpu.VMEM((N_G, 1), jnp.float32),
            pltpu.VMEM((1, 1), jnp.float32),
            pltpu.SemaphoreType.DMA((2,)),
            pltpu.SemaphoreType.DMA((2,)),
            pltpu.SemaphoreType.DMA((2,)),
        ],
    )
    energy, graph_repr, loss = pl.pallas_call(
        _fused_kernel,
        grid_spec=grid_spec,
        out_shape=[
            jax.ShapeDtypeStruct((N_G, 1), jnp.float32),
            jax.ShapeDtypeStruct((N_G, HID), jnp.float32),
            jax.ShapeDtypeStruct((1, 1), jnp.float32),
        ],
        compiler_params=pltpu.CompilerParams(
            dimension_semantics=("arbitrary",)),
    )(starts, H, Z, bid2d, nsig, noise, batch2d,
      W_enc, W_pos, W_out, W1, b1.reshape(1, HID), W2, b2.reshape(1, 1))
    return energy[:, 0], graph_repr, loss[0, 0]


# speedup vs baseline: 3.7079x; 1.4111x over previous
import jax
import jax.numpy as jnp
from jax.experimental import pallas as pl
from jax.experimental.pallas import tpu as pltpu

N_A = 320000
N_B = 32000
N_G = 64
HID = 128
BB = 256
AC = 1024
N_T = N_B // BB


def _silu(x):
    return x * jax.nn.sigmoid(x)


def _fused_kernel(starts_ref,
                  h_hbm, z_hbm, bid_hbm,
                  nsig_ref, noise_ref, batch_ref,
                  wenc_ref, wpos_ref, wout_ref, w1_ref, b1_ref, w2_ref, b2_ref,
                  energy_ref, graph_ref, loss_ref,
                  sums_ref, cnts_ref, hbuf, zbuf, bbuf,
                  gsum_ref, gcnt_ref, losss_ref,
                  sem_h, sem_z, sem_b):
    t = pl.program_id(0)
    b0 = t * BB
    astart = starts_ref[t]
    aend = starts_ref[t + 1]

    @pl.when(t == 0)
    def _():
        gsum_ref[...] = jnp.zeros_like(gsum_ref)
        gcnt_ref[...] = jnp.zeros_like(gcnt_ref)
        losss_ref[...] = jnp.zeros_like(losss_ref)

    sums_ref[...] = jnp.zeros_like(sums_ref)
    cnts_ref[...] = jnp.zeros_like(cnts_ref)

    n_sub = (aend - astart + AC - 1) // AC

    iota_bb = jax.lax.broadcasted_iota(jnp.int32, (AC, BB), 1)
    row_iota = jax.lax.broadcasted_iota(jnp.int32, (AC, 1), 0)
    ones_a = jnp.ones((AC, 1), jnp.float32)

    def start_copies(i, slot):
        base = astart + i * AC
        base_c = jnp.minimum(base, N_A - AC)
        pltpu.make_async_copy(h_hbm.at[pl.ds(base_c, AC), :],
                              hbuf.at[slot], sem_h.at[slot]).start()
        pltpu.make_async_copy(z_hbm.at[pl.ds(base_c, AC), :],
                              zbuf.at[slot], sem_z.at[slot]).start()
        pltpu.make_async_copy(bid_hbm.at[pl.ds(base_c, AC), :],
                              bbuf.at[slot], sem_b.at[slot]).start()

    def wait_copies(i, slot):
        base = astart + i * AC
        base_c = jnp.minimum(base, N_A - AC)
        pltpu.make_async_copy(h_hbm.at[pl.ds(base_c, AC), :],
                              hbuf.at[slot], sem_h.at[slot]).wait()
        pltpu.make_async_copy(z_hbm.at[pl.ds(base_c, AC), :],
                              zbuf.at[slot], sem_z.at[slot]).wait()
        pltpu.make_async_copy(bid_hbm.at[pl.ds(base_c, AC), :],
                              bbuf.at[slot], sem_b.at[slot]).wait()

    @pl.when(n_sub > 0)
    def _():
        start_copies(0, 0)

    def body(i, carry):
        slot = jax.lax.rem(i, 2)
        base = astart + i * AC
        base_c = jnp.minimum(base, N_A - AC)

        @pl.when(i + 1 < n_sub)
        def _():
            start_copies(i + 1, 1 - slot)

        wait_copies(i, slot)
        local = bbuf[slot] - b0
        valid = (base_c + row_iota) >= base
        oh = jnp.where(jnp.logical_and(local == iota_bb, valid),
                       1.0, 0.0).astype(jnp.float32)
        nadd = jax.lax.dot_general(oh, nsig_ref[...], (((1,), (0,)), ((), ())),
                                   preferred_element_type=jnp.float32, precision=jax.lax.Precision.HIGHEST)
        zp = zbuf[slot] + nadd
        pre = jnp.dot(hbuf[slot].astype(jnp.bfloat16),
                      wenc_ref[...].astype(jnp.bfloat16),
                      preferred_element_type=jnp.float32)
        zp16 = zp.astype(jnp.bfloat16).astype(jnp.float32)
        wpos16 = wpos_ref[...].astype(jnp.bfloat16).astype(jnp.float32)
        pre = (pre + zp16[:, 0:1] * wpos16[0:1, :]
               + zp16[:, 1:2] * wpos16[1:2, :]
               + zp16[:, 2:3] * wpos16[2:3, :])
        u = _silu(pre)
        oh16 = oh.astype(jnp.bfloat16)
        sums_ref[...] += jax.lax.dot_general(oh16, u.astype(jnp.bfloat16),
                                             (((0,), (0,)), ((), ())),
                                             preferred_element_type=jnp.float32)
        cnts_ref[...] += jax.lax.dot_general(oh16, ones_a.astype(jnp.bfloat16),
                                             (((0,), (0,)), ((), ())),
                                             preferred_element_type=jnp.float32)
        return carry

    jax.lax.fori_loop(0, n_sub, body, 0)

    cnt = cnts_ref[...]
    block_repr = sums_ref[...] / jnp.maximum(cnt, 1.0)
    iota_g = jax.lax.broadcasted_iota(jnp.int32, (BB, N_G), 1)
    ohg = jnp.where(batch_ref[...] == iota_g, 1.0, 0.0).astype(jnp.float32)
    gsum_ref[...] += jax.lax.dot_general(ohg, block_repr, (((0,), (0,)), ((), ())),
                                         preferred_element_type=jnp.float32, precision=jax.lax.Precision.HIGHEST)
    gcnt_ref[...] += jax.lax.dot_general(ohg, jnp.ones((BB, 1), jnp.float32),
                                         (((0,), (0,)), ((), ())),
                                         preferred_element_type=jnp.float32, precision=jax.lax.Precision.HIGHEST)
    pred = jnp.dot(block_repr, wout_ref[...], preferred_element_type=jnp.float32, precision=jax.lax.Precision.HIGHEST)
    score = -noise_ref[...] * jnp.where(cnt > 0.0, 1.0, 0.0)
    losss_ref[...] += jnp.reshape(jnp.sum((pred - score) ** 2), (1, 1))

    graph_repr = gsum_ref[...] / jnp.maximum(gcnt_ref[...], 1.0)
    graph_ref[...] = graph_repr
    e1 = _silu(jnp.dot(graph_repr.astype(jnp.bfloat16),
                       w1_ref[...].astype(jnp.bfloat16),
                       preferred_element_type=jnp.float32) + b1_ref[...])
    energy_ref[...] = jnp.dot(e1.astype(jnp.bfloat16),
                              w2_ref[...].astype(jnp.bfloat16),
                              preferred_element_type=jnp.float32) + b2_ref[...]
    loss_ref[...] = losss_ref[...] / (N_B * 3.0)


def kernel(Z, H, noise, sigmas, W_enc, W_pos, W_out, W1, b1, W2, b2,
           block_id, batch_id, noise_level):
    sig = sigmas[noise_level][batch_id]
    nsig = noise * sig[:, None]
    bounds = jnp.arange(N_T + 1, dtype=jnp.int32) * BB
    starts = jnp.searchsorted(block_id, bounds, side='left').astype(jnp.int32)
    bid2d = block_id.reshape(N_A, 1)
    batch2d = batch_id.reshape(N_B, 1)

    grid_spec = pltpu.PrefetchScalarGridSpec(
        num_scalar_prefetch=1,
        grid=(N_T,),
        in_specs=[
            pl.BlockSpec(memory_space=pl.ANY),
            pl.BlockSpec(memory_space=pl.ANY),
            pl.BlockSpec(memory_space=pl.ANY),
            pl.BlockSpec((BB, 3), lambda t, s: (t, 0)),
            pl.BlockSpec((BB, 3), lambda t, s: (t, 0)),
            pl.BlockSpec((BB, 1), lambda t, s: (t, 0)),
            pl.BlockSpec((HID, HID), lambda t, s: (0, 0)),
            pl.BlockSpec((3, HID), lambda t, s: (0, 0)),
            pl.BlockSpec((HID, 3), lambda t, s: (0, 0)),
            pl.BlockSpec((HID, HID), lambda t, s: (0, 0)),
            pl.BlockSpec((1, HID), lambda t, s: (0, 0)),
            pl.BlockSpec((HID, 1), lambda t, s: (0, 0)),
            pl.BlockSpec((1, 1), lambda t, s: (0, 0)),
        ],
        out_specs=[
            pl.BlockSpec((N_G, 1), lambda t, s: (0, 0)),
            pl.BlockSpec((N_G, HID), lambda t, s: (0, 0)),
            pl.BlockSpec((1, 1), lambda t, s: (0, 0)),
        ],
        scratch_shapes=[
            pltpu.VMEM((BB, HID), jnp.float32),
            pltpu.VMEM((BB, 1), jnp.float32),
            pltpu.VMEM((2, AC, HID), jnp.float32),
            pltpu.VMEM((2, AC, 3), jnp.float32),
            pltpu.VMEM((2, AC, 1), jnp.int32),
            pltpu.VMEM((N_G, HID), jnp.float32),
            pltpu.VMEM((N_G, 1), jnp.float32),
            pltpu.VMEM((1, 1), jnp.float32),
            pltpu.SemaphoreType.DMA((2,)),
            pltpu.SemaphoreType.DMA((2,)),
            pltpu.SemaphoreType.DMA((2,)),
        ],
    )
    energy, graph_repr, loss = pl.pallas_call(
        _fused_kernel,
        grid_spec=grid_spec,
        out_shape=[
            jax.ShapeDtypeStruct((N_G, 1), jnp.float32),
            jax.ShapeDtypeStruct((N_G, HID), jnp.float32),
            jax.ShapeDtypeStruct((1, 1), jnp.float32),
        ],
        compiler_params=pltpu.CompilerParams(
            dimension_semantics=("arbitrary",)),
    )(starts, H, Z, bid2d, nsig, noise, batch2d,
      W_enc, W_pos, W_out, W1, b1.reshape(1, HID), W2, b2.reshape(1, 1))
    return energy[:, 0], graph_repr, loss[0, 0]
